# baseline (speedup 1.0000x reference)
import functools

import jax
import jax.numpy as jnp
from jax import lax
from jax.experimental import pallas as pl
from jax.experimental.pallas import tpu as pltpu, tpu_sc as plsc

N = 8192
K = 8
NOFF = 27
CH = 128
EPS = 1e-5
ROWS = 128
MROWS = 256
NW = 32
CHUNK = 256


def _knn_kernel(pt_ref, ci_ref, idx_ref, nbr_ref, msk_ref):
    i = pl.program_id(0)
    p_all = pt_ref[0:3, :]
    p_blk = pt_ref[0:3, pl.ds(i * ROWS, ROWS)]
    x2a = p_all[0:1, :] * p_all[0:1, :]
    y2a = p_all[1:2, :] * p_all[1:2, :]
    z2a = p_all[2:3, :] * p_all[2:3, :]
    sq_all = (x2a + z2a) + y2a
    x2b = p_blk[0:1, :] * p_blk[0:1, :]
    y2b = p_blk[1:2, :] * p_blk[1:2, :]
    z2b = p_blk[2:3, :] * p_blk[2:3, :]
    sq_blk = (x2b + z2b) + y2b
    dot = jax.lax.dot_general(
        p_blk, p_all, (((0,), (0,)), ((), ())),
        preferred_element_type=jnp.float32)
    d2 = (jnp.transpose(sq_blk) + sq_all) - 2.0 * dot
    jidx = jax.lax.broadcasted_iota(jnp.int32, (ROWS, N), 1)
    big = jnp.int32(2**30)

    cols = [jax.lax.broadcasted_iota(jnp.int32, (ROWS, 1), 0) + i * ROWS]
    d2w = d2
    for _ in range(K):
        m = jnp.min(d2w, axis=1, keepdims=True)
        am = jnp.min(jnp.where(d2w == m, jidx, big), axis=1, keepdims=True)
        cols.append(am)
        d2w = jnp.where(jidx == am, jnp.float32(jnp.inf), d2w)
    cols.append(jnp.zeros((ROWS, 16 - K - 1), jnp.int32))
    idx_ref[...] = jnp.concatenate(cols, axis=1)

    bj = ci_ref[0:1, :]
    xj = ci_ref[1:2, :]
    yj = ci_ref[2:3, :]
    zj = ci_ref[3:4, :]
    bi = jnp.transpose(ci_ref[0:1, pl.ds(i * ROWS, ROWS)])
    xi = jnp.transpose(ci_ref[1:2, pl.ds(i * ROWS, ROWS)])
    yi = jnp.transpose(ci_ref[2:3, pl.ds(i * ROWS, ROWS)])
    zi = jnp.transpose(ci_ref[3:4, pl.ds(i * ROWS, ROWS)])
    dx = xj - xi
    dy = yj - yi
    dz = zj - zi
    one = jnp.int32(1)
    valid = ((bj == bi) & (jnp.abs(dx) <= one) & (jnp.abs(dy) <= one)
             & (jnp.abs(dz) <= one))
    code = jnp.where(valid, (dx + 1) * 9 + (dy + 1) * 3 + (dz + 1),
                     jnp.int32(-1))
    ncols = []
    for k in range(NOFF):
        ncols.append(jnp.min(jnp.where(code == k, jidx, big), axis=1,
                             keepdims=True))
    ncols.append(jnp.full((ROWS, 32 - NOFF), big, jnp.int32))
    nbr = jnp.concatenate(ncols, axis=1)
    msk = nbr < jnp.int32(N)
    nbr_ref[...] = jnp.where(msk, nbr, 0)
    msk_ref[...] = msk.astype(jnp.float32)


def _knn_pallas(coords):
    b = coords[:, 0].astype(jnp.float32)
    xyz = coords[:, 1:].astype(jnp.float32)
    p = xyz + b[:, None] * 1e4
    pt = jnp.zeros((8, N), jnp.float32).at[0:3, :].set(p.T)
    ci = jnp.zeros((8, N), jnp.int32)
    ci = ci.at[0, :].set(coords[:, 0])
    ci = ci.at[1:4, :].set(coords[:, 1:].T + 1)
    return pl.pallas_call(
        _knn_kernel,
        grid=(N // ROWS,),
        in_specs=[pl.BlockSpec((8, N), lambda i: (0, 0)),
                  pl.BlockSpec((8, N), lambda i: (0, 0))],
        out_specs=[pl.BlockSpec((ROWS, 16), lambda i: (i, 0)),
                   pl.BlockSpec((ROWS, 32), lambda i: (i, 0)),
                   pl.BlockSpec((ROWS, 32), lambda i: (i, 0))],
        out_shape=[jax.ShapeDtypeStruct((N, 16), jnp.int32),
                   jax.ShapeDtypeStruct((N, 32), jnp.int32),
                   jax.ShapeDtypeStruct((N, 32), jnp.float32)],
    )(pt, ci)


def _sc_gather(table, idx_flat):
    B = idx_flat.shape[0]
    b_per_w = B // NW
    n_chunks = b_per_w // CHUNK
    mesh = plsc.VectorSubcoreMesh(core_axis_name="c", subcore_axis_name="s")

    @functools.partial(
        pl.kernel, mesh=mesh,
        out_type=jax.ShapeDtypeStruct((B, CH), jnp.float32),
        scratch_types=[
            pltpu.VMEM((CHUNK,), jnp.int32),
            pltpu.VMEM((CHUNK, CH), jnp.float32),
            pltpu.SemaphoreType.DMA,
        ],
    )
    def gather_k(table_hbm, idx_hbm, out_hbm, idx_v, rows_v, sem):
        wid = lax.axis_index("s") * 2 + lax.axis_index("c")
        base = wid * b_per_w

        def body(ci, carry):
            off = base + ci * CHUNK
            pltpu.sync_copy(idx_hbm.at[pl.ds(off, CHUNK)], idx_v)
            pltpu.async_copy(table_hbm.at[idx_v], rows_v, sem).wait()
            pltpu.sync_copy(rows_v, out_hbm.at[pl.ds(off, CHUNK)])
            return carry

        lax.fori_loop(0, n_chunks, body, 0)

    return gather_k(table, idx_flat)


def _mlp1_kernel(g9_ref, W1_ref, b1_ref, hpre_ref, stat_ref):
    i = pl.program_id(0)
    x = g9_ref[...]
    hp = jnp.dot(x, W1_ref[...], preferred_element_type=jnp.float32)
    hp = hp + b1_ref[...]
    hpre_ref[...] = hp
    s = jnp.concatenate([
        jnp.sum(hp, axis=0, keepdims=True),
        jnp.sum(hp * hp, axis=0, keepdims=True),
        jnp.zeros((6, CH), jnp.float32)], axis=0)

    @pl.when(i == 0)
    def _():
        stat_ref[...] = jnp.zeros_like(stat_ref)

    stat_ref[...] += s


def _mlp1(g9, W1, b1):
    return pl.pallas_call(
        _mlp1_kernel,
        grid=(N // MROWS,),
        in_specs=[pl.BlockSpec((MROWS, (K + 1) * CH), lambda i: (i, 0)),
                  pl.BlockSpec(((K + 1) * CH, CH), lambda i: (0, 0)),
                  pl.BlockSpec((1, CH), lambda i: (0, 0))],
        out_specs=[pl.BlockSpec((MROWS, CH), lambda i: (i, 0)),
                   pl.BlockSpec((8, CH), lambda i: (0, 0))],
        out_shape=[jax.ShapeDtypeStruct((N, CH), jnp.float32),
                   jax.ShapeDtypeStruct((8, CH), jnp.float32)],
    )(g9, W1, b1.reshape(1, CH))


def _conv_kernel(gh_ref, msk_ref, W2_ref, b2_ref, s1_ref, g1_ref, be1_ref,
                 conv_ref, s2_ref, acc_ref):
    i = pl.program_id(0)
    k = pl.program_id(1)
    s1 = s1_ref[...]
    mean = s1[0:1, :] / N
    var = jnp.maximum(s1[1:2, :] / N - mean * mean, 0.0)
    scale = g1_ref[...] * jax.lax.rsqrt(var + EPS)
    shift = be1_ref[...] - mean * scale

    g = gh_ref[0]
    h = jnp.maximum(g * scale + shift, 0.0)
    m = msk_ref[0, 0, :]
    h = h * m[:, None]
    contrib = jnp.dot(h, W2_ref[0], preferred_element_type=jnp.float32)

    @pl.when(k == 0)
    def _():
        acc_ref[...] = b2_ref[...] + jnp.zeros((MROWS, CH), jnp.float32)

    acc_ref[...] += contrib

    @pl.when(k == NOFF - 1)
    def _():
        c = acc_ref[...]
        conv_ref[...] = c
        s = jnp.concatenate([
            jnp.sum(c, axis=0, keepdims=True),
            jnp.sum(c * c, axis=0, keepdims=True),
            jnp.zeros((6, CH), jnp.float32)], axis=0)

        @pl.when(i == 0)
        def _():
            s2_ref[...] = jnp.zeros_like(s2_ref)

        s2_ref[...] += s


def _conv(gh, msk_t, W2, b2, s1, g1, be1):
    return pl.pallas_call(
        _conv_kernel,
        grid=(N // MROWS, NOFF),
        in_specs=[pl.BlockSpec((1, MROWS, CH), lambda i, k: (k, i, 0)),
                  pl.BlockSpec((1, 1, MROWS), lambda i, k: (k, 0, i)),
                  pl.BlockSpec((1, CH, CH), lambda i, k: (k, 0, 0)),
                  pl.BlockSpec((1, CH), lambda i, k: (0, 0)),
                  pl.BlockSpec((8, CH), lambda i, k: (0, 0)),
                  pl.BlockSpec((1, CH), lambda i, k: (0, 0)),
                  pl.BlockSpec((1, CH), lambda i, k: (0, 0))],
        out_specs=[pl.BlockSpec((MROWS, CH), lambda i, k: (i, 0)),
                   pl.BlockSpec((8, CH), lambda i, k: (0, 0))],
        out_shape=[jax.ShapeDtypeStruct((N, CH), jnp.float32),
                   jax.ShapeDtypeStruct((8, CH), jnp.float32)],
        scratch_shapes=[pltpu.VMEM((MROWS, CH), jnp.float32)],
    )(gh, msk_t, W2, b2.reshape(1, CH), s1, g1.reshape(1, CH),
      be1.reshape(1, CH))


def _final_kernel(conv_ref, s2_ref, g2_ref, be2_ref, W3_ref, b3_ref, out_ref):
    s2 = s2_ref[...]
    mean = s2[0:1, :] / N
    var = jnp.maximum(s2[1:2, :] / N - mean * mean, 0.0)
    scale = g2_ref[...] * jax.lax.rsqrt(var + EPS)
    shift = be2_ref[...] - mean * scale
    h2 = jnp.maximum(conv_ref[...] * scale + shift, 0.0)
    out_ref[...] = jnp.dot(h2, W3_ref[...],
                           preferred_element_type=jnp.float32) + b3_ref[...]


def _final(conv, s2, g2, be2, W3, b3):
    OUT = W3.shape[1]
    return pl.pallas_call(
        _final_kernel,
        grid=(N // MROWS,),
        in_specs=[pl.BlockSpec((MROWS, CH), lambda i: (i, 0)),
                  pl.BlockSpec((8, CH), lambda i: (0, 0)),
                  pl.BlockSpec((1, CH), lambda i: (0, 0)),
                  pl.BlockSpec((1, CH), lambda i: (0, 0)),
                  pl.BlockSpec((CH, OUT), lambda i: (0, 0)),
                  pl.BlockSpec((1, OUT), lambda i: (0, 0))],
        out_specs=pl.BlockSpec((MROWS, OUT), lambda i: (i, 0)),
        out_shape=jax.ShapeDtypeStruct((N, OUT), jnp.float32),
    )(conv, s2, g2.reshape(1, CH), be2.reshape(1, CH), W3,
      b3.reshape(1, OUT))


def kernel(feats, coords, W1, b1, g1, beta1, W2, b2, g2, beta2, W3, b3):
    idx9, nbr, msk = _knn_pallas(coords)
    g9 = _sc_gather(feats, idx9[:, :K + 1].reshape(-1))
    g9 = g9.reshape(N, (K + 1) * CH)
    hpre, s1 = _mlp1(g9, W1, b1)
    gh = _sc_gather(hpre, nbr[:, :NOFF].T.reshape(-1))
    gh = gh.reshape(NOFF, N, CH)
    msk_t = msk[:, :NOFF].T.reshape(NOFF, 1, N)
    conv, s2 = _conv(gh, msk_t, W2, b2, s1, g1, beta1)
    return _final(conv, s2, g2, beta2, W3, b3)


# speedup vs baseline: 4.1041x; 4.1041x over previous
import functools

import jax
import jax.numpy as jnp
from jax import lax
from jax.experimental import pallas as pl
from jax.experimental.pallas import tpu as pltpu, tpu_sc as plsc

N = 8192
K = 8
NOFF = 27
CH = 128
EPS = 1e-5
ROWS = 128
MROWS = 256
NW = 32
CHUNK = 256


def _knn_kernel(pt_ref, ci_ref, idx_ref, nbr_ref, msk_ref):
    i = pl.program_id(0)
    p_all = pt_ref[0:3, :]
    p_blk = pt_ref[0:3, pl.ds(i * ROWS, ROWS)]
    x2a = p_all[0:1, :] * p_all[0:1, :]
    y2a = p_all[1:2, :] * p_all[1:2, :]
    z2a = p_all[2:3, :] * p_all[2:3, :]
    sq_all = (x2a + z2a) + y2a
    x2b = p_blk[0:1, :] * p_blk[0:1, :]
    y2b = p_blk[1:2, :] * p_blk[1:2, :]
    z2b = p_blk[2:3, :] * p_blk[2:3, :]
    sq_blk = (x2b + z2b) + y2b
    dot = jax.lax.dot_general(
        p_blk, p_all, (((0,), (0,)), ((), ())),
        preferred_element_type=jnp.float32)
    d2 = (jnp.transpose(sq_blk) + sq_all) - 2.0 * dot
    jidx = jax.lax.broadcasted_iota(jnp.int32, (ROWS, N), 1)
    big = jnp.int32(2**30)

    selfcol = jax.lax.broadcasted_iota(jnp.int32, (ROWS, 1), 0) + i * ROWS
    cols = [selfcol]
    d2w = d2
    for _ in range(K):
        m = jnp.min(d2w, axis=1, keepdims=True)
        am = jnp.min(jnp.where(d2w == m, jidx, big), axis=1, keepdims=True)
        cols.append(am)
        d2w = jnp.where(jidx == am, jnp.float32(jnp.inf), d2w)
    cols.append(jnp.zeros((ROWS, 16 - K - 1), jnp.int32))
    idx_ref[...] = jnp.concatenate(cols, axis=1)

    bj = ci_ref[0:1, :]
    xj = ci_ref[1:2, :]
    yj = ci_ref[2:3, :]
    zj = ci_ref[3:4, :]
    bi = jnp.transpose(ci_ref[0:1, pl.ds(i * ROWS, ROWS)])
    xi = jnp.transpose(ci_ref[1:2, pl.ds(i * ROWS, ROWS)])
    yi = jnp.transpose(ci_ref[2:3, pl.ds(i * ROWS, ROWS)])
    zi = jnp.transpose(ci_ref[3:4, pl.ds(i * ROWS, ROWS)])
    dx = xj - xi
    dy = yj - yi
    dz = zj - zi
    one = jnp.int32(1)
    valid = ((bj == bi) & (jnp.abs(dx) <= one) & (jnp.abs(dy) <= one)
             & (jnp.abs(dz) <= one))
    code = jnp.where(valid, (dx + 1) * 9 + (dy + 1) * 3 + (dz + 1),
                     jnp.int32(-1))
    ncols = []
    for k in range(NOFF):
        ncols.append(jnp.min(jnp.where(code == k, jidx, big), axis=1,
                             keepdims=True))
    ncols.append(jnp.full((ROWS, 32 - NOFF), big, jnp.int32))
    nbr = jnp.concatenate(ncols, axis=1)
    msk = nbr < jnp.int32(N)
    nbr_ref[...] = jnp.where(msk, nbr, selfcol)
    msk_ref[...] = msk.astype(jnp.float32)


def _knn_pallas(coords):
    b = coords[:, 0].astype(jnp.float32)
    xyz = coords[:, 1:].astype(jnp.float32)
    p = xyz + b[:, None] * 1e4
    pt = jnp.zeros((8, N), jnp.float32).at[0:3, :].set(p.T)
    ci = jnp.zeros((8, N), jnp.int32)
    ci = ci.at[0, :].set(coords[:, 0])
    ci = ci.at[1:4, :].set(coords[:, 1:].T + 1)
    return pl.pallas_call(
        _knn_kernel,
        grid=(N // ROWS,),
        in_specs=[pl.BlockSpec((8, N), lambda i: (0, 0)),
                  pl.BlockSpec((8, N), lambda i: (0, 0))],
        out_specs=[pl.BlockSpec((ROWS, 16), lambda i: (i, 0)),
                   pl.BlockSpec((ROWS, 32), lambda i: (i, 0)),
                   pl.BlockSpec((ROWS, 32), lambda i: (i, 0))],
        out_shape=[jax.ShapeDtypeStruct((N, 16), jnp.int32),
                   jax.ShapeDtypeStruct((N, 32), jnp.int32),
                   jax.ShapeDtypeStruct((N, 32), jnp.float32)],
    )(pt, ci)


def _sc_gather(table, idx_flat):
    B = idx_flat.shape[0]
    D = table.shape[1]
    dt = table.dtype
    b_per_w = B // NW
    n_chunks = b_per_w // CHUNK
    mesh = plsc.VectorSubcoreMesh(core_axis_name="c", subcore_axis_name="s")

    @functools.partial(
        pl.kernel, mesh=mesh,
        out_type=jax.ShapeDtypeStruct((B, D), dt),
        scratch_types=[
            pltpu.VMEM((CHUNK,), jnp.int32),
            pltpu.VMEM((CHUNK, D), dt),
            pltpu.SemaphoreType.DMA,
        ],
    )
    def gather_k(table_hbm, idx_hbm, out_hbm, idx_v, rows_v, sem):
        wid = lax.axis_index("s") * 2 + lax.axis_index("c")
        base = wid * b_per_w

        def body(ci, carry):
            off = base + ci * CHUNK
            pltpu.sync_copy(idx_hbm.at[pl.ds(off, CHUNK)], idx_v)
            pltpu.async_copy(table_hbm.at[idx_v], rows_v, sem).wait()
            pltpu.sync_copy(rows_v, out_hbm.at[pl.ds(off, CHUNK)])
            return carry

        lax.fori_loop(0, n_chunks, body, 0)

    return gather_k(table, idx_flat)


def _mlp1_kernel(g9_ref, W1_ref, b1_ref, hpre_ref, stat_ref):
    i = pl.program_id(0)
    x = g9_ref[...]
    hp = jnp.dot(x, W1_ref[...], preferred_element_type=jnp.float32)
    hp = hp + b1_ref[...]
    hpre_ref[...] = hp
    s = jnp.concatenate([
        jnp.sum(hp, axis=0, keepdims=True),
        jnp.sum(hp * hp, axis=0, keepdims=True),
        jnp.zeros((6, CH), jnp.float32)], axis=0)

    @pl.when(i == 0)
    def _():
        stat_ref[...] = jnp.zeros_like(stat_ref)

    stat_ref[...] += s


def _mlp1(g9, W1, b1):
    return pl.pallas_call(
        _mlp1_kernel,
        grid=(N // MROWS,),
        in_specs=[pl.BlockSpec((MROWS, (K + 1) * CH), lambda i: (i, 0)),
                  pl.BlockSpec(((K + 1) * CH, CH), lambda i: (0, 0)),
                  pl.BlockSpec((1, CH), lambda i: (0, 0))],
        out_specs=[pl.BlockSpec((MROWS, CH), lambda i: (i, 0)),
                   pl.BlockSpec((8, CH), lambda i: (0, 0))],
        out_shape=[jax.ShapeDtypeStruct((N, CH), jnp.float32),
                   jax.ShapeDtypeStruct((8, CH), jnp.float32)],
    )(g9, W1, b1.reshape(1, CH))


def _conv_kernel(gh_ref, msk_ref, W2_ref, b2_ref, s1_ref, g1_ref, be1_ref,
                 conv_ref, s2_ref, acc_ref):
    i = pl.program_id(0)
    k = pl.program_id(1)
    s1 = s1_ref[...]
    mean = s1[0:1, :] / N
    var = jnp.maximum(s1[1:2, :] / N - mean * mean, 0.0)
    scale = g1_ref[...] * jax.lax.rsqrt(var + EPS)
    shift = be1_ref[...] - mean * scale

    g = gh_ref[0]
    h = jnp.maximum(g * scale + shift, 0.0)
    m = msk_ref[0, 0, :]
    h = h * m[:, None]
    contrib = jnp.dot(h, W2_ref[0], preferred_element_type=jnp.float32)

    @pl.when(k == 0)
    def _():
        acc_ref[...] = b2_ref[...] + jnp.zeros((MROWS, CH), jnp.float32)

    acc_ref[...] += contrib

    @pl.when(k == NOFF - 1)
    def _():
        c = acc_ref[...]
        conv_ref[...] = c
        s = jnp.concatenate([
            jnp.sum(c, axis=0, keepdims=True),
            jnp.sum(c * c, axis=0, keepdims=True),
            jnp.zeros((6, CH), jnp.float32)], axis=0)

        @pl.when(i == 0)
        def _():
            s2_ref[...] = jnp.zeros_like(s2_ref)

        s2_ref[...] += s


def _conv(gh, msk_t, W2, b2, s1, g1, be1):
    return pl.pallas_call(
        _conv_kernel,
        grid=(N // MROWS, NOFF),
        in_specs=[pl.BlockSpec((1, MROWS, CH), lambda i, k: (k, i, 0)),
                  pl.BlockSpec((1, 1, MROWS), lambda i, k: (k, 0, i)),
                  pl.BlockSpec((1, CH, CH), lambda i, k: (k, 0, 0)),
                  pl.BlockSpec((1, CH), lambda i, k: (0, 0)),
                  pl.BlockSpec((8, CH), lambda i, k: (0, 0)),
                  pl.BlockSpec((1, CH), lambda i, k: (0, 0)),
                  pl.BlockSpec((1, CH), lambda i, k: (0, 0))],
        out_specs=[pl.BlockSpec((MROWS, CH), lambda i, k: (i, 0)),
                   pl.BlockSpec((8, CH), lambda i, k: (0, 0))],
        out_shape=[jax.ShapeDtypeStruct((N, CH), jnp.float32),
                   jax.ShapeDtypeStruct((8, CH), jnp.float32)],
        scratch_shapes=[pltpu.VMEM((MROWS, CH), jnp.float32)],
    )(gh, msk_t, W2, b2.reshape(1, CH), s1, g1.reshape(1, CH),
      be1.reshape(1, CH))


def _final_kernel(conv_ref, s2_ref, g2_ref, be2_ref, W3_ref, b3_ref, out_ref):
    s2 = s2_ref[...]
    mean = s2[0:1, :] / N
    var = jnp.maximum(s2[1:2, :] / N - mean * mean, 0.0)
    scale = g2_ref[...] * jax.lax.rsqrt(var + EPS)
    shift = be2_ref[...] - mean * scale
    h2 = jnp.maximum(conv_ref[...] * scale + shift, 0.0)
    out_ref[...] = jnp.dot(h2, W3_ref[...],
                           preferred_element_type=jnp.float32) + b3_ref[...]


def _final(conv, s2, g2, be2, W3, b3):
    OUT = W3.shape[1]
    return pl.pallas_call(
        _final_kernel,
        grid=(N // MROWS,),
        in_specs=[pl.BlockSpec((MROWS, CH), lambda i: (i, 0)),
                  pl.BlockSpec((8, CH), lambda i: (0, 0)),
                  pl.BlockSpec((1, CH), lambda i: (0, 0)),
                  pl.BlockSpec((1, CH), lambda i: (0, 0)),
                  pl.BlockSpec((CH, OUT), lambda i: (0, 0)),
                  pl.BlockSpec((1, OUT), lambda i: (0, 0))],
        out_specs=pl.BlockSpec((MROWS, OUT), lambda i: (i, 0)),
        out_shape=jax.ShapeDtypeStruct((N, OUT), jnp.float32),
    )(conv, s2, g2.reshape(1, CH), be2.reshape(1, CH), W3,
      b3.reshape(1, OUT))


def kernel(feats, coords, W1, b1, g1, beta1, W2, b2, g2, beta2, W3, b3):
    idx9, nbr, msk = _knn_pallas(coords)
    g9 = _sc_gather(feats, idx9[:, :K + 1].reshape(-1))
    g9 = g9.reshape(N, (K + 1) * CH)
    hpre, s1 = _mlp1(g9, W1, b1)
    gh = _sc_gather(hpre, nbr[:, :NOFF].T.reshape(-1))
    gh = gh.reshape(NOFF, N, CH)
    msk_t = msk[:, :NOFF].T.reshape(NOFF, 1, N)
    conv, s2 = _conv(gh, msk_t, W2, b2, s1, g1, beta1)
    return _final(conv, s2, g2, beta2, W3, b3)


# speedup vs baseline: 6.7724x; 1.6501x over previous
import functools

import jax
import jax.numpy as jnp
from jax import lax
from jax.experimental import pallas as pl
from jax.experimental.pallas import tpu as pltpu, tpu_sc as plsc

N = 8192
K = 8
NOFF = 27
CH = 128
EPS = 1e-5
ROWS = 128
MROWS = 256
NW = 32
CHUNK = 256


def _knn_kernel(pt_ref, idx_ref):
    i = pl.program_id(0)
    p_all = pt_ref[0:3, :]
    p_blk = pt_ref[0:3, pl.ds(i * ROWS, ROWS)]
    x2a = p_all[0:1, :] * p_all[0:1, :]
    y2a = p_all[1:2, :] * p_all[1:2, :]
    z2a = p_all[2:3, :] * p_all[2:3, :]
    sq_all = (x2a + z2a) + y2a
    x2b = p_blk[0:1, :] * p_blk[0:1, :]
    y2b = p_blk[1:2, :] * p_blk[1:2, :]
    z2b = p_blk[2:3, :] * p_blk[2:3, :]
    sq_blk = (x2b + z2b) + y2b
    dot = jax.lax.dot_general(
        p_blk, p_all, (((0,), (0,)), ((), ())),
        preferred_element_type=jnp.float32)
    d2 = (jnp.transpose(sq_blk) + sq_all) - 2.0 * dot
    jidx = jax.lax.broadcasted_iota(jnp.int32, (ROWS, N), 1)
    big = jnp.int32(2**30)

    selfcol = jax.lax.broadcasted_iota(jnp.int32, (ROWS, 1), 0) + i * ROWS
    cols = [selfcol]
    d2w = d2
    for _ in range(K):
        m = jnp.min(d2w, axis=1, keepdims=True)
        am = jnp.min(jnp.where(d2w == m, jidx, big), axis=1, keepdims=True)
        cols.append(am)
        d2w = jnp.where(jidx == am, jnp.float32(jnp.inf), d2w)
    cols.append(jnp.zeros((ROWS, 16 - K - 1), jnp.int32))
    idx_ref[...] = jnp.concatenate(cols, axis=1)


def _knn_pallas(coords):
    b = coords[:, 0].astype(jnp.float32)
    xyz = coords[:, 1:].astype(jnp.float32)
    p = xyz + b[:, None] * 1e4
    pt = jnp.zeros((8, N), jnp.float32).at[0:3, :].set(p.T)
    return pl.pallas_call(
        _knn_kernel,
        grid=(N // ROWS,),
        in_specs=[pl.BlockSpec((8, N), lambda i: (0, 0))],
        out_specs=pl.BlockSpec((ROWS, 16), lambda i: (i, 0)),
        out_shape=jax.ShapeDtypeStruct((N, 16), jnp.int32),
    )(pt)


NB = 64
CAP = 256


def _pos_kernel(ci_ref, pos_ref, cp_ref):
    b = ci_ref[0:1, :]
    brow = jax.lax.broadcasted_iota(jnp.int32, (NB, 1), 0)
    oh = (b == brow).astype(jnp.float32)
    cum = oh
    sh = 1
    while sh < N:
        cum = cum + jnp.concatenate(
            [jnp.zeros((NB, sh), jnp.float32), cum[:, :N - sh]], axis=1)
        sh *= 2
    rank = jnp.sum(oh * cum, axis=0, keepdims=True) - 1.0
    pos_ref[...] = (b * CAP) + rank.astype(jnp.int32)
    cxyz = ci_ref[1:4, :].astype(jnp.float32)
    orig = jax.lax.broadcasted_iota(jnp.int32, (1, N), 1).astype(jnp.float32)
    zero = jnp.zeros((124, N), jnp.float32)
    cp_ref[...] = jnp.transpose(jnp.concatenate([cxyz, orig, zero], axis=0))


def _pos_cp(coords):
    ci = jnp.zeros((8, N), jnp.int32)
    ci = ci.at[0, :].set(coords[:, 0])
    ci = ci.at[1:4, :].set(coords[:, 1:].T + 1)
    return pl.pallas_call(
        _pos_kernel,
        grid=(1,),
        in_specs=[pl.BlockSpec((8, N), lambda i: (0, 0))],
        out_specs=[pl.BlockSpec((1, N), lambda i: (0, 0)),
                   pl.BlockSpec((N, 128), lambda i: (0, 0))],
        out_shape=[jax.ShapeDtypeStruct((1, N), jnp.int32),
                   jax.ShapeDtypeStruct((N, 128), jnp.float32)],
    )(ci)


def _conv_match_kernel(ccp_ref, mem_ref, combo_ref):
    cp = ccp_ref[0]
    mm = mem_ref[0, 0:1, :] + mem_ref[0, 1:2, :] - 1
    valid_j = mm >= 0
    orig_j = cp[:, 3:4].astype(jnp.int32)
    jorig = jnp.transpose(orig_j)
    jorigB = jorig + jnp.zeros((CAP, 1), jnp.int32)
    big = jnp.int32(2**30)
    cx_i = cp[:, 0:1]
    cy_i = cp[:, 1:2]
    cz_i = cp[:, 2:3]
    dx = jnp.transpose(cx_i) - cx_i
    dy = jnp.transpose(cy_i) - cy_i
    dz = jnp.transpose(cz_i) - cz_i
    onef = jnp.float32(1.0)
    okd = ((jnp.abs(dx) <= onef) & (jnp.abs(dy) <= onef)
           & (jnp.abs(dz) <= onef) & valid_j)
    code = (dx + 1.0) * 9.0 + (dy + 1.0) * 3.0 + (dz + 1.0)
    ncols = []
    for k in range(NOFF):
        cond = okd & (code == jnp.float32(k))
        ncols.append(jnp.min(jnp.where(cond, jorigB, big), axis=1,
                             keepdims=True))
    ncols.append(jnp.full((CAP, 32 - NOFF), big, jnp.int32))
    nbr = jnp.concatenate(ncols, axis=1)
    mskv = nbr < jnp.int32(N)
    nbr = jnp.where(mskv, nbr, orig_j)
    combo_ref[...] = jnp.concatenate(
        [nbr, mskv.astype(jnp.int32), jnp.zeros((CAP, 64), jnp.int32)],
        axis=1)


def _conv_match(ccp, mem2):
    return pl.pallas_call(
        _conv_match_kernel,
        grid=(NB,),
        in_specs=[pl.BlockSpec((1, CAP, 128), lambda i: (i, 0, 0)),
                  pl.BlockSpec((1, 2, CAP), lambda i: (i, 0, 0))],
        out_specs=pl.BlockSpec((CAP, 128), lambda i: (i, 0)),
        out_shape=jax.ShapeDtypeStruct((NB * CAP, 128), jnp.int32),
    )(ccp.reshape(NB, CAP, 128), mem2.reshape(2, NB, CAP).transpose(1, 0, 2))


def _sc_gather(table, idx_flat):
    B = idx_flat.shape[0]
    D = table.shape[1]
    dt = table.dtype
    b_per_w = B // NW
    n_chunks = b_per_w // CHUNK
    mesh = plsc.VectorSubcoreMesh(core_axis_name="c", subcore_axis_name="s")

    @functools.partial(
        pl.kernel, mesh=mesh,
        out_type=jax.ShapeDtypeStruct((B, D), dt),
        scratch_types=[
            pltpu.VMEM((CHUNK,), jnp.int32),
            pltpu.VMEM((CHUNK, D), dt),
            pltpu.SemaphoreType.DMA,
        ],
    )
    def gather_k(table_hbm, idx_hbm, out_hbm, idx_v, rows_v, sem):
        wid = lax.axis_index("s") * 2 + lax.axis_index("c")
        base = wid * b_per_w

        def body(ci, carry):
            off = base + ci * CHUNK
            pltpu.sync_copy(idx_hbm.at[pl.ds(off, CHUNK)], idx_v)
            pltpu.async_copy(table_hbm.at[idx_v], rows_v, sem).wait()
            pltpu.sync_copy(rows_v, out_hbm.at[pl.ds(off, CHUNK)])
            return carry

        lax.fori_loop(0, n_chunks, body, 0)

    return gather_k(table, idx_flat)


def _mlp1_kernel(g9_ref, W1_ref, b1_ref, hpre_ref, stat_ref):
    i = pl.program_id(0)
    x = g9_ref[...]
    hp = jnp.dot(x, W1_ref[...], preferred_element_type=jnp.float32)
    hp = hp + b1_ref[...]
    hpre_ref[...] = hp
    s = jnp.concatenate([
        jnp.sum(hp, axis=0, keepdims=True),
        jnp.sum(hp * hp, axis=0, keepdims=True),
        jnp.zeros((6, CH), jnp.float32)], axis=0)

    @pl.when(i == 0)
    def _():
        stat_ref[...] = jnp.zeros_like(stat_ref)

    stat_ref[...] += s


def _mlp1(g9, W1, b1):
    return pl.pallas_call(
        _mlp1_kernel,
        grid=(N // MROWS,),
        in_specs=[pl.BlockSpec((MROWS, (K + 1) * CH), lambda i: (i, 0)),
                  pl.BlockSpec(((K + 1) * CH, CH), lambda i: (0, 0)),
                  pl.BlockSpec((1, CH), lambda i: (0, 0))],
        out_specs=[pl.BlockSpec((MROWS, CH), lambda i: (i, 0)),
                   pl.BlockSpec((8, CH), lambda i: (0, 0))],
        out_shape=[jax.ShapeDtypeStruct((N, CH), jnp.float32),
                   jax.ShapeDtypeStruct((8, CH), jnp.float32)],
    )(g9, W1, b1.reshape(1, CH))


def _conv_kernel(gh_ref, msk_ref, W2_ref, b2_ref, s1_ref, g1_ref, be1_ref,
                 conv_ref, s2_ref, acc_ref):
    i = pl.program_id(0)
    k = pl.program_id(1)
    s1 = s1_ref[...]
    mean = s1[0:1, :] / N
    var = jnp.maximum(s1[1:2, :] / N - mean * mean, 0.0)
    scale = g1_ref[...] * jax.lax.rsqrt(var + EPS)
    shift = be1_ref[...] - mean * scale

    g = gh_ref[0]
    h = jnp.maximum(g * scale + shift, 0.0)
    m = msk_ref[0, 0, :]
    h = h * m[:, None]
    contrib = jnp.dot(h, W2_ref[0], preferred_element_type=jnp.float32)

    @pl.when(k == 0)
    def _():
        acc_ref[...] = b2_ref[...] + jnp.zeros((MROWS, CH), jnp.float32)

    acc_ref[...] += contrib

    @pl.when(k == NOFF - 1)
    def _():
        c = acc_ref[...]
        conv_ref[...] = c
        s = jnp.concatenate([
            jnp.sum(c, axis=0, keepdims=True),
            jnp.sum(c * c, axis=0, keepdims=True),
            jnp.zeros((6, CH), jnp.float32)], axis=0)

        @pl.when(i == 0)
        def _():
            s2_ref[...] = jnp.zeros_like(s2_ref)

        s2_ref[...] += s


def _conv(gh, msk_t, W2, b2, s1, g1, be1):
    return pl.pallas_call(
        _conv_kernel,
        grid=(N // MROWS, NOFF),
        in_specs=[pl.BlockSpec((1, MROWS, CH), lambda i, k: (k, i, 0)),
                  pl.BlockSpec((1, 1, MROWS), lambda i, k: (k, 0, i)),
                  pl.BlockSpec((1, CH, CH), lambda i, k: (k, 0, 0)),
                  pl.BlockSpec((1, CH), lambda i, k: (0, 0)),
                  pl.BlockSpec((8, CH), lambda i, k: (0, 0)),
                  pl.BlockSpec((1, CH), lambda i, k: (0, 0)),
                  pl.BlockSpec((1, CH), lambda i, k: (0, 0))],
        out_specs=[pl.BlockSpec((MROWS, CH), lambda i, k: (i, 0)),
                   pl.BlockSpec((8, CH), lambda i, k: (0, 0))],
        out_shape=[jax.ShapeDtypeStruct((N, CH), jnp.float32),
                   jax.ShapeDtypeStruct((8, CH), jnp.float32)],
        scratch_shapes=[pltpu.VMEM((MROWS, CH), jnp.float32)],
    )(gh, msk_t, W2, b2.reshape(1, CH), s1, g1.reshape(1, CH),
      be1.reshape(1, CH))


def _final_kernel(conv_ref, s2_ref, g2_ref, be2_ref, W3_ref, b3_ref, out_ref):
    s2 = s2_ref[...]
    mean = s2[0:1, :] / N
    var = jnp.maximum(s2[1:2, :] / N - mean * mean, 0.0)
    scale = g2_ref[...] * jax.lax.rsqrt(var + EPS)
    shift = be2_ref[...] - mean * scale
    h2 = jnp.maximum(conv_ref[...] * scale + shift, 0.0)
    out_ref[...] = jnp.dot(h2, W3_ref[...],
                           preferred_element_type=jnp.float32) + b3_ref[...]


def _final(conv, s2, g2, be2, W3, b3):
    OUT = W3.shape[1]
    return pl.pallas_call(
        _final_kernel,
        grid=(N // MROWS,),
        in_specs=[pl.BlockSpec((MROWS, CH), lambda i: (i, 0)),
                  pl.BlockSpec((8, CH), lambda i: (0, 0)),
                  pl.BlockSpec((1, CH), lambda i: (0, 0)),
                  pl.BlockSpec((1, CH), lambda i: (0, 0)),
                  pl.BlockSpec((CH, OUT), lambda i: (0, 0)),
                  pl.BlockSpec((1, OUT), lambda i: (0, 0))],
        out_specs=pl.BlockSpec((MROWS, OUT), lambda i: (i, 0)),
        out_shape=jax.ShapeDtypeStruct((N, OUT), jnp.float32),
    )(conv, s2, g2.reshape(1, CH), be2.reshape(1, CH), W3,
      b3.reshape(1, OUT))


def kernel(feats, coords, W1, b1, g1, beta1, W2, b2, g2, beta2, W3, b3):
    idx9 = _knn_pallas(coords)
    pos, cp = _pos_cp(coords)
    pos = pos.reshape(N)
    members = jnp.full((NB * CAP,), 0, jnp.int32).at[pos].set(
        jnp.arange(1, N + 1, dtype=jnp.int32))
    ccp = _sc_gather(cp, jnp.maximum(members - 1, 0))
    mem2 = jnp.stack([members, jnp.zeros_like(members)])
    combo_s = _conv_match(ccp, mem2)
    combo = _sc_gather(combo_s, pos)
    nbr = combo[:, :32]
    msk = combo[:, 32:64].astype(jnp.float32)
    g9 = _sc_gather(feats, idx9[:, :K + 1].reshape(-1))
    g9 = g9.reshape(N, (K + 1) * CH)
    hpre, s1 = _mlp1(g9, W1, b1)
    gh = _sc_gather(hpre, nbr[:, :NOFF].T.reshape(-1))
    gh = gh.reshape(NOFF, N, CH)
    msk_t = msk[:, :NOFF].T.reshape(NOFF, 1, N)
    conv, s2 = _conv(gh, msk_t, W2, b2, s1, g1, beta1)
    return _final(conv, s2, g2, beta2, W3, b3)
